# Initial kernel scaffold; baseline (speedup 1.0000x reference)
import jax
import jax.numpy as jnp
from jax.experimental import pallas as pl


def kernel(x, edge_index, edge_weight, W_in, b_in, W_out, b_out, gamma, beta):
    raise NotImplementedError("write your pallas kernel here")


# speedup vs baseline: 8.8261x; 8.8261x over previous
import functools

import jax
import jax.numpy as jnp
from jax import lax
from jax.experimental import pallas as pl
from jax.experimental.pallas import tpu as pltpu
from jax.experimental.pallas import tpu_sc as plsc

N = 10000
E = 160000
R = 8
D = 256
H = 256
NL = 3
ALPHA = 0.75

K = 2 * R
NS = 16
NC = 2
NB = 88
SB = 8
NST = NB // SB
CHUNK = NB * 128
ETP = NS * CHUNK
N2 = 10240
FH = 128

_mesh = plsc.VectorSubcoreMesh(core_axis_name="c", subcore_axis_name="s")


def _zero_vmem(ref, nwords):
    zeros = jnp.zeros((16,), ref.dtype)

    def body(i, _):
        ref[pl.ds(i * 16, 16)] = zeros
        return 0

    lax.fori_loop(0, nwords // 16, body, 0)


def _zero_vmem2(ref, m, w):
    zeros = jnp.zeros((16,), ref.dtype)

    def body(i, _):
        for q in range(w // 16):
            ref[i, pl.ds(q * 16, 16)] = zeros
        return 0

    lax.fori_loop(0, m, body, 0)


def _bcast_lane(v, t):
    idx = jnp.full((16,), t, jnp.int32)
    return lax.gather(
        v, idx[:, None],
        dimension_numbers=lax.GatherDimensionNumbers(
            offset_dims=(), collapsed_slice_dims=(0,), start_index_map=(0,)),
        slice_sizes=(1,),
        mode=lax.GatherScatterMode.PROMISE_IN_BOUNDS)


@functools.partial(
    pl.kernel,
    out_type=jax.ShapeDtypeStruct((NC, (K // NC) * N2), jnp.float32),
    mesh=_mesh,
    scratch_types=[
        pltpu.VMEM_SHARED(((K // NC) * N2,), jnp.float32),
        pltpu.VMEM((5120,), jnp.float32),
        pltpu.VMEM((NB, 128), jnp.int32),
        pltpu.VMEM((NB, 128), jnp.float32),
    ],
)
def _p1_deg(si_hbm, wv_hbm, out_hbm, deg_sh, zbuf, si_v, wv_v):
    c = lax.axis_index("c")
    s = lax.axis_index("s")
    _zero_vmem(zbuf, 5120)
    pltpu.sync_copy(zbuf, deg_sh.at[pl.ds(s * 5120, 5120)])
    plsc.subcore_barrier()

    def per_k(kk, _):
        k = c * (K // NC) + kk
        pltpu.sync_copy(si_hbm.at[k, s], si_v)
        pltpu.sync_copy(wv_hbm.at[k, s], wv_v)
        off = kk * N2

        def add_off(b, _):
            for q in range(8):
                si_v[b, pl.ds(q * 16, 16)] = si_v[b, pl.ds(q * 16, 16)] + off
            return 0

        lax.fori_loop(0, NB, add_off, 0)

        def scat(b, _):
            pltpu.sync_copy(wv_v.at[b], deg_sh.at[si_v.at[b]], add=True)
            return 0

        lax.fori_loop(0, NB, scat, 0)
        return 0

    lax.fori_loop(0, K // NC, per_k, 0)
    plsc.subcore_barrier()

    @pl.when(s == 0)
    def _():
        pltpu.sync_copy(deg_sh, out_hbm.at[c])


def _p2_body(deg_ref, out_ref):
    d = deg_ref[...]
    dis = jnp.where(d > 0.0, lax.rsqrt(d), 0.0)
    rows = lax.broadcasted_iota(jnp.int32, (K, N2), 0)
    a = jnp.where(rows % 2 == 0, ALPHA ** 0.5, (1.0 - ALPHA) ** 0.5)
    out_ref[...] = dis * a


_p2_dis = pl.pallas_call(
    _p2_body,
    out_shape=jax.ShapeDtypeStruct((K, N2), jnp.float32),
)


@functools.partial(
    pl.kernel,
    out_type=(
        jax.ShapeDtypeStruct((K, NS, NB, 128), jnp.float32),
        jax.ShapeDtypeStruct((K, NS, NB, 128), jnp.int32),
    ),
    mesh=_mesh,
    scratch_types=[
        pltpu.VMEM((NB, 128), jnp.int32),
        pltpu.VMEM((NB, 128), jnp.int32),
        pltpu.VMEM((NB, 128), jnp.float32),
        pltpu.VMEM((NB, 128), jnp.float32),
        pltpu.VMEM((NB, 128), jnp.int32),
        pltpu.VMEM((128,), jnp.float32),
        pltpu.VMEM((128,), jnp.float32),
    ],
)
def _p3_norm(gi_hbm, si_hbm, wv_hbm, dis_hbm, nrm_hbm, gf_hbm,
             gi_v, si_v, wv_v, nrm_v, gf_v, dgb, dsb):
    c = lax.axis_index("c")
    s = lax.axis_index("s")

    def per_k(kk, _):
        k = c * (K // NC) + kk
        pltpu.sync_copy(gi_hbm.at[k, s], gi_v)
        pltpu.sync_copy(si_hbm.at[k, s], si_v)
        pltpu.sync_copy(wv_hbm.at[k, s], wv_v)
        goff = k * N
        doff = k * N2

        def add_off(b, _):
            for q in range(8):
                sl = pl.ds(q * 16, 16)
                g = gi_v[b, sl]
                gf_v[b, sl] = g + goff
                gi_v[b, sl] = g + doff
                si_v[b, sl] = si_v[b, sl] + doff
            return 0

        lax.fori_loop(0, NB, add_off, 0)

        def per_batch(b, _):
            pltpu.sync_copy(dis_hbm.at[gi_v.at[b]], dgb)
            pltpu.sync_copy(dis_hbm.at[si_v.at[b]], dsb)
            for q in range(8):
                sl = pl.ds(q * 16, 16)
                nrm_v[b, sl] = dgb[sl] * wv_v[b, sl] * dsb[sl]
            return 0

        lax.fori_loop(0, NB, per_batch, 0)
        pltpu.sync_copy(nrm_v, nrm_hbm.at[k, s])
        pltpu.sync_copy(gf_v, gf_hbm.at[k, s])
        return 0

    lax.fori_loop(0, K // NC, per_k, 0)


def _k4_body(x_ref, w_ref, out_ref):
    out_ref[0, 0] = jnp.dot(x_ref[...], w_ref[0],
                            preferred_element_type=jnp.float32,
                            precision=lax.Precision.HIGHEST)


BN = 1000

_k4_matmul = pl.pallas_call(
    _k4_body,
    grid=(N // BN, K, NC),
    in_specs=[
        pl.BlockSpec((BN, D), lambda nb, k, c: (nb, 0)),
        pl.BlockSpec((1, D, FH), lambda nb, k, c: (k, 0, c)),
    ],
    out_specs=pl.BlockSpec((1, 1, BN, FH), lambda nb, k, c: (c, k, nb, 0)),
    out_shape=jax.ShapeDtypeStruct((NC, K, N, FH), jnp.float32),
)


@functools.partial(
    pl.kernel,
    out_type=jax.ShapeDtypeStruct((NC, N, FH), jnp.float32),
    mesh=_mesh,
    scratch_types=[
        pltpu.VMEM_SHARED((N, FH), jnp.float32),
        pltpu.VMEM((2, 128, FH), jnp.float32),
        pltpu.VMEM((SB, 128), jnp.int32),
        pltpu.VMEM((SB, 128), jnp.int32),
        pltpu.VMEM((SB, 128), jnp.float32),
        pltpu.SemaphoreType.DMA,
        pltpu.SemaphoreType.DMA,
    ],
)
def _k5_agg(hw_hbm, gf_hbm, si_hbm, nrm_hbm, out_hbm,
            acc_sh, rows, gf_v, si_v, nr_v, gsem, ssem):
    c = lax.axis_index("c")
    s = lax.axis_index("s")

    _zero_vmem2(rows.at[0], 128, FH)
    for i in range(5):
        j = s + 16 * i

        @pl.when(j < 78)
        def _():
            pltpu.sync_copy(rows.at[0], acc_sh.at[pl.ds(j * 128, 128)])

    @pl.when(s == 15)
    def _():
        pltpu.sync_copy(rows.at[0, pl.ds(0, 16)], acc_sh.at[pl.ds(9984, 16)])

    plsc.subcore_barrier()

    coff = c * (K * N)

    def per_k(k, _):
        def per_stage(st, _):
            b0 = st * SB
            pltpu.sync_copy(gf_hbm.at[k, s, pl.ds(b0, SB)], gf_v)
            pltpu.sync_copy(si_hbm.at[k, s, pl.ds(b0, SB)], si_v)
            pltpu.sync_copy(nrm_hbm.at[k, s, pl.ds(b0, SB)], nr_v)

            def add_off(b, _):
                for q in range(8):
                    sl = pl.ds(q * 16, 16)
                    gf_v[b, sl] = gf_v[b, sl] + coff
                return 0

            lax.fori_loop(0, SB, add_off, 0)

            pltpu.async_copy(hw_hbm.at[gf_v.at[0]], rows.at[0], gsem)

            def per_batch(b, _):
                slot = lax.rem(b, 2)
                nslot = lax.rem(b + 1, 2)
                pltpu.make_async_copy(hw_hbm.at[gf_v.at[b]], rows.at[slot],
                                      gsem).wait()

                @pl.when(b >= 1)
                def _():
                    pltpu.make_async_copy(rows.at[nslot],
                                          acc_sh.at[si_v.at[b]], ssem).wait()

                @pl.when(b + 1 < SB)
                def _():
                    pltpu.async_copy(hw_hbm.at[gf_v.at[b + 1]], rows.at[nslot],
                                     gsem)

                def scale_q(q, _):
                    nrm16 = nr_v[b, pl.ds(q * 16, 16)]

                    def scale_t(t, _):
                        nv = _bcast_lane(nrm16, t)
                        e = q * 16 + t
                        for f in range(8):
                            sl = pl.ds(f * 16, 16)
                            rows[slot, e, sl] = rows[slot, e, sl] * nv
                        return 0

                    lax.fori_loop(0, 16, scale_t, 0)
                    return 0

                lax.fori_loop(0, 8, scale_q, 0)
                pltpu.async_copy(rows.at[slot], acc_sh.at[si_v.at[b]], ssem,
                                 add=True)
                return 0

            lax.fori_loop(0, SB, per_batch, 0)
            pltpu.make_async_copy(rows.at[0], acc_sh.at[si_v.at[0]],
                                  ssem).wait()
            return 0

        lax.fori_loop(0, NST, per_stage, 0)
        return 0

    lax.fori_loop(0, K, per_k, 0)
    plsc.subcore_barrier()
    base = s * 624
    pltpu.sync_copy(acc_sh.at[pl.ds(base, 624)], out_hbm.at[c, pl.ds(base, 624)])

    @pl.when(s == 15)
    def _():
        pltpu.sync_copy(acc_sh.at[pl.ds(9984, 16)],
                        out_hbm.at[c, pl.ds(9984, 16)])


def _k6_body(acc_ref, bi_ref, bo_ref, g_ref, bt_ref, out_ref):
    z = jnp.concatenate([acc_ref[0], acc_ref[1]], axis=1)
    bsum = (jnp.sum(bi_ref[...], axis=0, keepdims=True) * ALPHA
            + jnp.sum(bo_ref[...], axis=0, keepdims=True) * (1.0 - ALPHA))
    z = z + bsum
    mu = jnp.mean(z, axis=-1, keepdims=True)
    zc = z - mu
    var = jnp.mean(zc * zc, axis=-1, keepdims=True)
    y = zc * lax.rsqrt(var + 1e-5) * g_ref[...] + bt_ref[...]
    out_ref[...] = jnp.maximum(y, 0.0)


_k6_ln = pl.pallas_call(
    _k6_body,
    grid=(N // BN,),
    in_specs=[
        pl.BlockSpec((NC, BN, FH), lambda nb: (0, nb, 0)),
        pl.BlockSpec((R, H), lambda nb: (0, 0)),
        pl.BlockSpec((R, H), lambda nb: (0, 0)),
        pl.BlockSpec((1, H), lambda nb: (0, 0)),
        pl.BlockSpec((1, H), lambda nb: (0, 0)),
    ],
    out_specs=pl.BlockSpec((BN, H), lambda nb: (nb, 0)),
    out_shape=jax.ShapeDtypeStruct((N, H), jnp.float32),
)


def kernel(x, edge_index, edge_weight, W_in, b_in, W_out, b_out, gamma, beta):
    pad = ETP - E - N
    loops_i = jnp.broadcast_to(jnp.arange(N, dtype=jnp.int32), (K, N))
    pad_i = jnp.broadcast_to(jnp.arange(pad, dtype=jnp.int32) % N, (K, pad))
    gi = jnp.concatenate([edge_index.reshape(K, E), loops_i, pad_i], axis=1)
    si = jnp.concatenate(
        [edge_index[:, ::-1, :].reshape(K, E), loops_i, pad_i], axis=1)
    wv = jnp.concatenate(
        [jnp.repeat(edge_weight, 2, axis=0), jnp.ones((K, N), jnp.float32),
         jnp.zeros((K, pad), jnp.float32)], axis=1)
    gi4 = gi.reshape(K, NS, NB, 128)
    si4 = si.reshape(K, NS, NB, 128)
    wv4 = wv.reshape(K, NS, NB, 128)

    deg = _p1_deg(si4, wv4).reshape(K, N2)
    dis2 = _p2_dis(deg)
    nrm4, gf4 = _p3_norm(gi4, si4, wv4, dis2.reshape(K * N2))

    h = x
    for l in range(NL):
        wl = jnp.stack([W_in[l], W_out[l]], axis=1).reshape(K, D, H)
        hw = _k4_matmul(h, wl).reshape(NC * K * N, FH)
        acc2 = _k5_agg(hw, gf4, si4, nrm4)
        h = _k6_ln(acc2, b_in[l], b_out[l], gamma[l][None], beta[l][None])
    return h


# speedup vs baseline: 9.8950x; 1.1211x over previous
import functools

import jax
import jax.numpy as jnp
from jax import lax
from jax.experimental import pallas as pl
from jax.experimental.pallas import tpu as pltpu
from jax.experimental.pallas import tpu_sc as plsc

N = 10000
E = 160000
R = 8
D = 256
H = 256
NL = 3
ALPHA = 0.75

K = 2 * R
NS = 16
NC = 2
NB = 88
SB = 8
NST = NB // SB
CHUNK = NB * 128
ETP = NS * CHUNK
N2 = 10240
FH = 128

_mesh = plsc.VectorSubcoreMesh(core_axis_name="c", subcore_axis_name="s")


def _zero_vmem(ref, nwords):
    zeros = jnp.zeros((16,), ref.dtype)

    def body(i, _):
        ref[pl.ds(i * 16, 16)] = zeros
        return 0

    lax.fori_loop(0, nwords // 16, body, 0)


def _zero_vmem2(ref, m, w):
    zeros = jnp.zeros((16,), ref.dtype)

    def body(i, _):
        for q in range(w // 16):
            ref[i, pl.ds(q * 16, 16)] = zeros
        return 0

    lax.fori_loop(0, m, body, 0)


def _bcast_lane(v, t):
    idx = jnp.full((16,), t, jnp.int32)
    return lax.gather(
        v, idx[:, None],
        dimension_numbers=lax.GatherDimensionNumbers(
            offset_dims=(), collapsed_slice_dims=(0,), start_index_map=(0,)),
        slice_sizes=(1,),
        mode=lax.GatherScatterMode.PROMISE_IN_BOUNDS)


@functools.partial(
    pl.kernel,
    out_type=jax.ShapeDtypeStruct((NC, (K // NC) * N2), jnp.float32),
    mesh=_mesh,
    scratch_types=[
        pltpu.VMEM_SHARED(((K // NC) * N2,), jnp.float32),
        pltpu.VMEM((5120,), jnp.float32),
        pltpu.VMEM((NB, 128), jnp.int32),
        pltpu.VMEM((NB, 128), jnp.float32),
    ],
)
def _p1_deg(si_hbm, wv_hbm, out_hbm, deg_sh, zbuf, si_v, wv_v):
    c = lax.axis_index("c")
    s = lax.axis_index("s")
    _zero_vmem(zbuf, 5120)
    pltpu.sync_copy(zbuf, deg_sh.at[pl.ds(s * 5120, 5120)])
    plsc.subcore_barrier()

    def per_k(kk, _):
        k = c * (K // NC) + kk
        pltpu.sync_copy(si_hbm.at[k, s], si_v)
        pltpu.sync_copy(wv_hbm.at[k, s], wv_v)
        off = kk * N2

        def add_off(b, _):
            for q in range(8):
                si_v[b, pl.ds(q * 16, 16)] = si_v[b, pl.ds(q * 16, 16)] + off
            return 0

        lax.fori_loop(0, NB, add_off, 0)

        def scat(b, _):
            pltpu.sync_copy(wv_v.at[b], deg_sh.at[si_v.at[b]], add=True)
            return 0

        lax.fori_loop(0, NB, scat, 0)
        return 0

    lax.fori_loop(0, K // NC, per_k, 0)
    plsc.subcore_barrier()

    @pl.when(s == 0)
    def _():
        pltpu.sync_copy(deg_sh, out_hbm.at[c])


def _p2_body(deg_ref, out_ref):
    d = deg_ref[...]
    dis = jnp.where(d > 0.0, lax.rsqrt(d), 0.0)
    rows = lax.broadcasted_iota(jnp.int32, (K, N2), 0)
    a = jnp.where(rows % 2 == 0, ALPHA ** 0.5, (1.0 - ALPHA) ** 0.5)
    out_ref[...] = dis * a


_p2_dis = pl.pallas_call(
    _p2_body,
    out_shape=jax.ShapeDtypeStruct((K, N2), jnp.float32),
)


@functools.partial(
    pl.kernel,
    out_type=(
        jax.ShapeDtypeStruct((NS, K, NB, 128), jnp.float32),
        jax.ShapeDtypeStruct((NS, K, NB, 128), jnp.int32),
        jax.ShapeDtypeStruct((NS, K, NB, 128), jnp.int32),
    ),
    mesh=_mesh,
    scratch_types=[
        pltpu.VMEM((NB, 128), jnp.int32),
        pltpu.VMEM((NB, 128), jnp.int32),
        pltpu.VMEM((NB, 128), jnp.float32),
        pltpu.VMEM((NB, 128), jnp.float32),
        pltpu.VMEM((NB, 128), jnp.int32),
        pltpu.VMEM((2, 128), jnp.float32),
        pltpu.VMEM((2, 128), jnp.float32),
        pltpu.SemaphoreType.DMA,
    ],
)
def _p3_norm(gi_hbm, si_hbm, wv_hbm, dis_hbm, nrm_hbm, gf_hbm, sit_hbm,
             gi_v, si_v, wv_v, nrm_v, gf_v, dgb, dsb, dsem):
    c = lax.axis_index("c")
    s = lax.axis_index("s")

    def per_k(kk, _):
        k = c * (K // NC) + kk
        pltpu.sync_copy(gi_hbm.at[k, s], gi_v)
        pltpu.sync_copy(si_hbm.at[k, s], si_v)
        pltpu.sync_copy(wv_hbm.at[k, s], wv_v)
        pltpu.sync_copy(si_v, sit_hbm.at[s, k])
        goff = k * N
        doff = k * N2

        def add_off(b, _):
            for q in range(8):
                sl = pl.ds(q * 16, 16)
                g = gi_v[b, sl]
                gf_v[b, sl] = g + goff
                gi_v[b, sl] = g + doff
                si_v[b, sl] = si_v[b, sl] + doff
            return 0

        lax.fori_loop(0, NB, add_off, 0)

        pltpu.async_copy(dis_hbm.at[gi_v.at[0]], dgb.at[0], dsem)
        pltpu.async_copy(dis_hbm.at[si_v.at[0]], dsb.at[0], dsem)

        def per_batch(b, _):
            sb = lax.rem(b, 2)
            nsb = lax.rem(b + 1, 2)
            pltpu.make_async_copy(dis_hbm.at[gi_v.at[b]], dgb.at[sb],
                                  dsem).wait()
            pltpu.make_async_copy(dis_hbm.at[si_v.at[b]], dsb.at[sb],
                                  dsem).wait()

            @pl.when(b + 1 < NB)
            def _():
                pltpu.async_copy(dis_hbm.at[gi_v.at[b + 1]], dgb.at[nsb], dsem)
                pltpu.async_copy(dis_hbm.at[si_v.at[b + 1]], dsb.at[nsb], dsem)

            for q in range(8):
                sl = pl.ds(q * 16, 16)
                nrm_v[b, sl] = dgb[sb, sl] * wv_v[b, sl] * dsb[sb, sl]
            return 0

        lax.fori_loop(0, NB, per_batch, 0)
        pltpu.sync_copy(nrm_v, nrm_hbm.at[s, k])
        pltpu.sync_copy(gf_v, gf_hbm.at[s, k])
        return 0

    lax.fori_loop(0, K // NC, per_k, 0)


def _k4_body(x_ref, w_ref, out_ref):
    out_ref[0, 0] = jnp.dot(x_ref[...], w_ref[0],
                            preferred_element_type=jnp.float32)


BN = 1000

_k4_matmul = pl.pallas_call(
    _k4_body,
    grid=(N // BN, K, NC),
    in_specs=[
        pl.BlockSpec((BN, D), lambda nb, k, c: (nb, 0)),
        pl.BlockSpec((1, D, FH), lambda nb, k, c: (k, 0, c)),
    ],
    out_specs=pl.BlockSpec((1, 1, BN, FH), lambda nb, k, c: (c, k, nb, 0)),
    out_shape=jax.ShapeDtypeStruct((NC, K, N, FH), jnp.float32),
)


NSTT = K * NB // SB


@functools.partial(
    pl.kernel,
    out_type=jax.ShapeDtypeStruct((NC, N, FH), jnp.float32),
    mesh=_mesh,
    scratch_types=[
        pltpu.VMEM_SHARED((N, FH), jnp.float32),
        pltpu.VMEM((2, 128, FH), jnp.float32),
        pltpu.VMEM((2, SB, 128), jnp.int32),
        pltpu.VMEM((2, SB, 128), jnp.int32),
        pltpu.VMEM((2, SB, 128), jnp.float32),
        pltpu.SemaphoreType.DMA,
        pltpu.SemaphoreType.DMA,
        pltpu.SemaphoreType.DMA,
    ],
)
def _k5_agg(hw_hbm, gf_hbm, si_hbm, nrm_hbm, out_hbm,
            acc_sh, rows, gf_v, si_v, nr_v, gsem, ssem, stsem):
    c = lax.axis_index("c")
    s = lax.axis_index("s")

    _zero_vmem2(rows.at[0], 128, FH)
    for i in range(5):
        j = s + 16 * i

        @pl.when(j < 78)
        def _():
            pltpu.sync_copy(rows.at[0], acc_sh.at[pl.ds(j * 128, 128)])

    @pl.when(s == 15)
    def _():
        pltpu.sync_copy(rows.at[0, pl.ds(0, 16)], acc_sh.at[pl.ds(9984, 16)])

    plsc.subcore_barrier()

    coff = c * (K * N)

    pltpu.async_copy(gf_hbm.at[s, pl.ds(0, SB)], gf_v.at[0], stsem)
    pltpu.async_copy(si_hbm.at[s, pl.ds(0, SB)], si_v.at[0], stsem)
    pltpu.async_copy(nrm_hbm.at[s, pl.ds(0, SB)], nr_v.at[0], stsem)

    def per_slice(st, _):
        isl = lax.rem(st, 2)
        insl = lax.rem(st + 1, 2)
        pltpu.make_async_copy(gf_hbm.at[s, pl.ds(0, SB)], gf_v.at[isl],
                              stsem).wait()
        pltpu.make_async_copy(si_hbm.at[s, pl.ds(0, SB)], si_v.at[isl],
                              stsem).wait()
        pltpu.make_async_copy(nrm_hbm.at[s, pl.ds(0, SB)], nr_v.at[isl],
                              stsem).wait()

        @pl.when(st + 1 < NSTT)
        def _():
            b1 = (st + 1) * SB
            pltpu.async_copy(gf_hbm.at[s, pl.ds(b1, SB)], gf_v.at[insl], stsem)
            pltpu.async_copy(si_hbm.at[s, pl.ds(b1, SB)], si_v.at[insl], stsem)
            pltpu.async_copy(nrm_hbm.at[s, pl.ds(b1, SB)], nr_v.at[insl],
                             stsem)

        def add_off(b, _):
            for q in range(8):
                sl = pl.ds(q * 16, 16)
                gf_v[isl, b, sl] = gf_v[isl, b, sl] + coff
            return 0

        lax.fori_loop(0, SB, add_off, 0)

        pltpu.async_copy(hw_hbm.at[gf_v.at[isl, 0]], rows.at[0], gsem)

        def per_batch(b, _):
            slot = lax.rem(b, 2)
            nslot = lax.rem(b + 1, 2)
            pltpu.make_async_copy(hw_hbm.at[gf_v.at[isl, b]], rows.at[slot],
                                  gsem).wait()

            @pl.when(b >= 1)
            def _():
                pltpu.make_async_copy(rows.at[nslot],
                                      acc_sh.at[si_v.at[isl, b]], ssem).wait()

            @pl.when(b + 1 < SB)
            def _():
                pltpu.async_copy(hw_hbm.at[gf_v.at[isl, b + 1]],
                                 rows.at[nslot], gsem)

            def scale_q(q, _):
                nrm16 = nr_v[isl, b, pl.ds(q * 16, 16)]

                def scale_t(t, _):
                    for dt in range(2):
                        e = q * 16 + t * 2 + dt
                        nv = _bcast_lane(nrm16, t * 2 + dt)
                        for f in range(8):
                            sl = pl.ds(f * 16, 16)
                            rows[slot, e, sl] = rows[slot, e, sl] * nv
                    return 0

                lax.fori_loop(0, 8, scale_t, 0)
                return 0

            lax.fori_loop(0, 8, scale_q, 0)
            pltpu.async_copy(rows.at[slot], acc_sh.at[si_v.at[isl, b]], ssem,
                             add=True)
            return 0

        lax.fori_loop(0, SB, per_batch, 0)
        pltpu.make_async_copy(rows.at[0], acc_sh.at[si_v.at[isl, 0]],
                              ssem).wait()
        return 0

    lax.fori_loop(0, NSTT, per_slice, 0)
    plsc.subcore_barrier()
    base = s * 624
    pltpu.sync_copy(acc_sh.at[pl.ds(base, 624)], out_hbm.at[c, pl.ds(base, 624)])

    @pl.when(s == 15)
    def _():
        pltpu.sync_copy(acc_sh.at[pl.ds(9984, 16)],
                        out_hbm.at[c, pl.ds(9984, 16)])


def _k6_body(acc_ref, bi_ref, bo_ref, g_ref, bt_ref, out_ref):
    z = jnp.concatenate([acc_ref[0], acc_ref[1]], axis=1)
    bsum = (jnp.sum(bi_ref[...], axis=0, keepdims=True) * ALPHA
            + jnp.sum(bo_ref[...], axis=0, keepdims=True) * (1.0 - ALPHA))
    z = z + bsum
    mu = jnp.mean(z, axis=-1, keepdims=True)
    zc = z - mu
    var = jnp.mean(zc * zc, axis=-1, keepdims=True)
    y = zc * lax.rsqrt(var + 1e-5) * g_ref[...] + bt_ref[...]
    out_ref[...] = jnp.maximum(y, 0.0)


_k6_ln = pl.pallas_call(
    _k6_body,
    grid=(N // BN,),
    in_specs=[
        pl.BlockSpec((NC, BN, FH), lambda nb: (0, nb, 0)),
        pl.BlockSpec((R, H), lambda nb: (0, 0)),
        pl.BlockSpec((R, H), lambda nb: (0, 0)),
        pl.BlockSpec((1, H), lambda nb: (0, 0)),
        pl.BlockSpec((1, H), lambda nb: (0, 0)),
    ],
    out_specs=pl.BlockSpec((BN, H), lambda nb: (nb, 0)),
    out_shape=jax.ShapeDtypeStruct((N, H), jnp.float32),
)


def kernel(x, edge_index, edge_weight, W_in, b_in, W_out, b_out, gamma, beta):
    pad = ETP - E - N
    loops_i = jnp.broadcast_to(jnp.arange(N, dtype=jnp.int32), (K, N))
    pad_i = jnp.broadcast_to(jnp.arange(pad, dtype=jnp.int32) % N, (K, pad))
    gi = jnp.concatenate([edge_index.reshape(K, E), loops_i, pad_i], axis=1)
    si = jnp.concatenate(
        [edge_index[:, ::-1, :].reshape(K, E), loops_i, pad_i], axis=1)
    wv = jnp.concatenate(
        [jnp.repeat(edge_weight, 2, axis=0), jnp.ones((K, N), jnp.float32),
         jnp.zeros((K, pad), jnp.float32)], axis=1)
    gi4 = gi.reshape(K, NS, NB, 128)
    si4 = si.reshape(K, NS, NB, 128)
    wv4 = wv.reshape(K, NS, NB, 128)

    deg = _p1_deg(si4, wv4).reshape(K, N2)
    dis2 = _p2_dis(deg)
    nrmT, gfT, siT = _p3_norm(gi4, si4, wv4, dis2.reshape(K * N2))
    nrmF = nrmT.reshape(NS, K * NB, 128)
    gfF = gfT.reshape(NS, K * NB, 128)
    siF = siT.reshape(NS, K * NB, 128)

    h = x
    for l in range(NL):
        wl = jnp.stack([W_in[l], W_out[l]], axis=1).reshape(K, D, H)
        hw = _k4_matmul(h, wl).reshape(NC * K * N, FH)
        acc2 = _k5_agg(hw, gfF, siF, nrmF)
        h = _k6_ln(acc2, b_in[l], b_out[l], gamma[l][None], beta[l][None])
    return h


# speedup vs baseline: 11.0025x; 1.1119x over previous
import functools

import jax
import jax.numpy as jnp
from jax import lax
from jax.experimental import pallas as pl
from jax.experimental.pallas import tpu as pltpu
from jax.experimental.pallas import tpu_sc as plsc

N = 10000
E = 160000
R = 8
D = 256
H = 256
NL = 3
ALPHA = 0.75

K = 2 * R
NS = 16
NC = 2
NB = 84
SB = 8
CHUNK = NB * 128
ETP = NS * CHUNK
N2 = 10240
FH = 128

_mesh = plsc.VectorSubcoreMesh(core_axis_name="c", subcore_axis_name="s")


def _zero_vmem(ref, nwords):
    zeros = jnp.zeros((16,), ref.dtype)

    def body(i, _):
        ref[pl.ds(i * 16, 16)] = zeros
        return 0

    lax.fori_loop(0, nwords // 16, body, 0)


def _zero_vmem2(ref, m, w):
    zeros = jnp.zeros((16,), ref.dtype)

    def body(i, _):
        for q in range(w // 16):
            ref[i, pl.ds(q * 16, 16)] = zeros
        return 0

    lax.fori_loop(0, m, body, 0)


def _bcast_lane(v, t):
    idx = jnp.full((16,), t, jnp.int32)
    return lax.gather(
        v, idx[:, None],
        dimension_numbers=lax.GatherDimensionNumbers(
            offset_dims=(), collapsed_slice_dims=(0,), start_index_map=(0,)),
        slice_sizes=(1,),
        mode=lax.GatherScatterMode.PROMISE_IN_BOUNDS)


@functools.partial(
    pl.kernel,
    out_type=jax.ShapeDtypeStruct((NC, (K // NC) * N2), jnp.float32),
    mesh=_mesh,
    scratch_types=[
        pltpu.VMEM_SHARED(((K // NC) * N2,), jnp.float32),
        pltpu.VMEM((5120,), jnp.float32),
        pltpu.VMEM((NB, 128), jnp.int32),
        pltpu.VMEM((NB, 128), jnp.float32),
    ],
)
def _p1_deg(si_hbm, wv_hbm, out_hbm, deg_sh, zbuf, si_v, wv_v):
    c = lax.axis_index("c")
    s = lax.axis_index("s")
    _zero_vmem(zbuf, 5120)
    pltpu.sync_copy(zbuf, deg_sh.at[pl.ds(s * 5120, 5120)])
    plsc.subcore_barrier()

    def per_k(kk, _):
        k = c * (K // NC) + kk
        pltpu.sync_copy(si_hbm.at[k, s], si_v)
        pltpu.sync_copy(wv_hbm.at[k, s], wv_v)
        off = kk * N2

        def add_off(b, _):
            for q in range(8):
                si_v[b, pl.ds(q * 16, 16)] = si_v[b, pl.ds(q * 16, 16)] + off
            return 0

        lax.fori_loop(0, NB, add_off, 0)

        def scat(b, _):
            pltpu.sync_copy(wv_v.at[b], deg_sh.at[si_v.at[b]], add=True)
            return 0

        lax.fori_loop(0, NB, scat, 0)
        return 0

    lax.fori_loop(0, K // NC, per_k, 0)
    plsc.subcore_barrier()

    @pl.when(s == 0)
    def _():
        pltpu.sync_copy(deg_sh, out_hbm.at[c])


def _p2_body(deg_ref, out_ref):
    d = deg_ref[...]
    dis = jnp.where(d > 0.0, lax.rsqrt(d), 0.0)
    rows = lax.broadcasted_iota(jnp.int32, (K, N2), 0)
    a = jnp.where(rows % 2 == 0, ALPHA ** 0.5, (1.0 - ALPHA) ** 0.5)
    out_ref[...] = dis * a


_p2_dis = pl.pallas_call(
    _p2_body,
    out_shape=jax.ShapeDtypeStruct((K, N2), jnp.float32),
)


@functools.partial(
    pl.kernel,
    out_type=(
        jax.ShapeDtypeStruct((NS, K, NB, 128), jnp.float32),
        jax.ShapeDtypeStruct((NS, K, NB, 128), jnp.int32),
        jax.ShapeDtypeStruct((NS, K, NB, 128), jnp.int32),
    ),
    mesh=_mesh,
    scratch_types=[
        pltpu.VMEM((NB, 128), jnp.int32),
        pltpu.VMEM((NB, 128), jnp.int32),
        pltpu.VMEM((NB, 128), jnp.float32),
        pltpu.VMEM((NB, 128), jnp.float32),
        pltpu.VMEM((NB, 128), jnp.int32),
        pltpu.VMEM((2, 128), jnp.float32),
        pltpu.VMEM((2, 128), jnp.float32),
        pltpu.VMEM_SHARED((K * N2,), jnp.float32),
        pltpu.SemaphoreType.DMA,
    ],
)
def _p3_norm(gi_hbm, si_hbm, wv_hbm, dis_hbm, nrm_hbm, gf_hbm, sit_hbm,
             gi_v, si_v, wv_v, nrm_v, gf_v, dgb, dsb, dis_sh, dsem):
    c = lax.axis_index("c")
    s = lax.axis_index("s")

    @pl.when(s == 0)
    def _():
        pltpu.sync_copy(dis_hbm, dis_sh)

    plsc.subcore_barrier()

    def per_k(kk, _):
        k = c * (K // NC) + kk
        pltpu.sync_copy(gi_hbm.at[k, s], gi_v)
        pltpu.sync_copy(si_hbm.at[k, s], si_v)
        pltpu.sync_copy(wv_hbm.at[k, s], wv_v)
        pltpu.sync_copy(si_v, sit_hbm.at[s, k])
        goff = k * N
        doff = k * N2

        def add_off(b, _):
            for q in range(8):
                sl = pl.ds(q * 16, 16)
                g = gi_v[b, sl]
                gf_v[b, sl] = g + goff
                gi_v[b, sl] = g + doff
                si_v[b, sl] = si_v[b, sl] + doff
            return 0

        lax.fori_loop(0, NB, add_off, 0)

        pltpu.async_copy(dis_sh.at[gi_v.at[0]], dgb.at[0], dsem)
        pltpu.async_copy(dis_sh.at[si_v.at[0]], dsb.at[0], dsem)

        def per_batch(b, _):
            sb = lax.rem(b, 2)
            nsb = lax.rem(b + 1, 2)
            pltpu.make_async_copy(dis_sh.at[gi_v.at[b]], dgb.at[sb],
                                  dsem).wait()
            pltpu.make_async_copy(dis_sh.at[si_v.at[b]], dsb.at[sb],
                                  dsem).wait()

            @pl.when(b + 1 < NB)
            def _():
                pltpu.async_copy(dis_sh.at[gi_v.at[b + 1]], dgb.at[nsb], dsem)
                pltpu.async_copy(dis_sh.at[si_v.at[b + 1]], dsb.at[nsb], dsem)

            for q in range(8):
                sl = pl.ds(q * 16, 16)
                nrm_v[b, sl] = dgb[sb, sl] * wv_v[b, sl] * dsb[sb, sl]
            return 0

        lax.fori_loop(0, NB, per_batch, 0)
        pltpu.sync_copy(nrm_v, nrm_hbm.at[s, k])
        pltpu.sync_copy(gf_v, gf_hbm.at[s, k])
        return 0

    lax.fori_loop(0, K // NC, per_k, 0)


def _k4_body(x_ref, w_ref, out_ref):
    out_ref[0, 0] = jnp.dot(x_ref[...], w_ref[0],
                            preferred_element_type=jnp.float32)


BN = 1000

_k4_matmul = pl.pallas_call(
    _k4_body,
    grid=(N // BN, K, NC),
    in_specs=[
        pl.BlockSpec((BN, D), lambda nb, k, c: (nb, 0)),
        pl.BlockSpec((1, D, FH), lambda nb, k, c: (k, 0, c)),
    ],
    out_specs=pl.BlockSpec((1, 1, BN, FH), lambda nb, k, c: (c, k, nb, 0)),
    out_shape=jax.ShapeDtypeStruct((NC, K, N, FH), jnp.float32),
)


NSTT = K * NB // SB


@functools.partial(
    pl.kernel,
    out_type=jax.ShapeDtypeStruct((NC, N, FH), jnp.float32),
    mesh=_mesh,
    scratch_types=[
        pltpu.VMEM_SHARED((N, FH), jnp.float32),
        pltpu.VMEM((2, 128, FH), jnp.float32),
        pltpu.VMEM((SB, 128), jnp.int32),
        pltpu.VMEM((SB, 128), jnp.int32),
        pltpu.VMEM((SB, 128), jnp.float32),
        pltpu.SemaphoreType.DMA,
        pltpu.SemaphoreType.DMA,
        pltpu.SemaphoreType.DMA,
    ],
)
def _k5_agg(hw_hbm, gf_hbm, si_hbm, nrm_hbm, out_hbm,
            acc_sh, rows, gf_v, si_v, nr_v, gsem, ssem, stsem):
    c = lax.axis_index("c")
    s = lax.axis_index("s")

    _zero_vmem2(rows.at[0], 128, FH)
    for i in range(5):
        j = s + 16 * i

        @pl.when(j < 78)
        def _():
            pltpu.sync_copy(rows.at[0], acc_sh.at[pl.ds(j * 128, 128)])

    @pl.when(s == 15)
    def _():
        pltpu.sync_copy(rows.at[0, pl.ds(0, 16)], acc_sh.at[pl.ds(9984, 16)])

    plsc.subcore_barrier()

    coff = c * (K * N)

    def per_slice(st, _):
        b0 = st * SB
        pltpu.async_copy(gf_hbm.at[s, pl.ds(b0, SB)], gf_v, stsem)
        pltpu.async_copy(si_hbm.at[s, pl.ds(b0, SB)], si_v, stsem)
        pltpu.async_copy(nrm_hbm.at[s, pl.ds(b0, SB)], nr_v, stsem)
        pltpu.make_async_copy(gf_hbm.at[s, pl.ds(b0, SB)], gf_v, stsem).wait()
        pltpu.make_async_copy(si_hbm.at[s, pl.ds(b0, SB)], si_v, stsem).wait()
        pltpu.make_async_copy(nrm_hbm.at[s, pl.ds(b0, SB)], nr_v, stsem).wait()

        def add_off(b, _):
            for q in range(8):
                sl = pl.ds(q * 16, 16)
                gf_v[b, sl] = gf_v[b, sl] + coff
            return 0

        lax.fori_loop(0, SB, add_off, 0)

        pltpu.async_copy(hw_hbm.at[gf_v.at[0]], rows.at[0], gsem)

        def per_batch(b, _):
            slot = lax.rem(b, 2)
            nslot = lax.rem(b + 1, 2)
            pltpu.make_async_copy(hw_hbm.at[gf_v.at[b]], rows.at[slot],
                                  gsem).wait()

            @pl.when(b >= 1)
            def _():
                pltpu.make_async_copy(rows.at[nslot],
                                      acc_sh.at[si_v.at[b]], ssem).wait()

            @pl.when(b + 1 < SB)
            def _():
                pltpu.async_copy(hw_hbm.at[gf_v.at[b + 1]], rows.at[nslot],
                                 gsem)

            def scale_q(q, _):
                nrm16 = nr_v[b, pl.ds(q * 16, 16)]

                def scale_t(t, _):
                    for dt in range(2):
                        e = q * 16 + t * 2 + dt
                        nv = _bcast_lane(nrm16, t * 2 + dt)
                        for f in range(8):
                            sl = pl.ds(f * 16, 16)
                            rows[slot, e, sl] = rows[slot, e, sl] * nv
                    return 0

                lax.fori_loop(0, 8, scale_t, 0)
                return 0

            lax.fori_loop(0, 8, scale_q, 0)
            pltpu.async_copy(rows.at[slot], acc_sh.at[si_v.at[b]], ssem,
                             add=True)
            return 0

        lax.fori_loop(0, SB, per_batch, 0)
        pltpu.make_async_copy(rows.at[0], acc_sh.at[si_v.at[0]], ssem).wait()
        return 0

    lax.fori_loop(0, NSTT, per_slice, 0)
    plsc.subcore_barrier()
    base = s * 624
    pltpu.sync_copy(acc_sh.at[pl.ds(base, 624)], out_hbm.at[c, pl.ds(base, 624)])

    @pl.when(s == 15)
    def _():
        pltpu.sync_copy(acc_sh.at[pl.ds(9984, 16)],
                        out_hbm.at[c, pl.ds(9984, 16)])


def _k6_body(acc_ref, bi_ref, bo_ref, g_ref, bt_ref, out_ref):
    z = jnp.concatenate([acc_ref[0], acc_ref[1]], axis=1)
    bsum = (jnp.sum(bi_ref[...], axis=0, keepdims=True) * ALPHA
            + jnp.sum(bo_ref[...], axis=0, keepdims=True) * (1.0 - ALPHA))
    z = z + bsum
    mu = jnp.mean(z, axis=-1, keepdims=True)
    zc = z - mu
    var = jnp.mean(zc * zc, axis=-1, keepdims=True)
    y = zc * lax.rsqrt(var + 1e-5) * g_ref[...] + bt_ref[...]
    out_ref[...] = jnp.maximum(y, 0.0)


_k6_ln = pl.pallas_call(
    _k6_body,
    grid=(N // BN,),
    in_specs=[
        pl.BlockSpec((NC, BN, FH), lambda nb: (0, nb, 0)),
        pl.BlockSpec((R, H), lambda nb: (0, 0)),
        pl.BlockSpec((R, H), lambda nb: (0, 0)),
        pl.BlockSpec((1, H), lambda nb: (0, 0)),
        pl.BlockSpec((1, H), lambda nb: (0, 0)),
    ],
    out_specs=pl.BlockSpec((BN, H), lambda nb: (nb, 0)),
    out_shape=jax.ShapeDtypeStruct((N, H), jnp.float32),
)


def kernel(x, edge_index, edge_weight, W_in, b_in, W_out, b_out, gamma, beta):
    pad = ETP - E - N
    loops_i = jnp.broadcast_to(jnp.arange(N, dtype=jnp.int32), (K, N))
    pad_i = jnp.broadcast_to(jnp.arange(pad, dtype=jnp.int32) % N, (K, pad))
    gi = jnp.concatenate([edge_index.reshape(K, E), loops_i, pad_i], axis=1)
    si = jnp.concatenate(
        [edge_index[:, ::-1, :].reshape(K, E), loops_i, pad_i], axis=1)
    wv = jnp.concatenate(
        [jnp.repeat(edge_weight, 2, axis=0), jnp.ones((K, N), jnp.float32),
         jnp.zeros((K, pad), jnp.float32)], axis=1)
    gi4 = gi.reshape(K, NS, NB, 128)
    si4 = si.reshape(K, NS, NB, 128)
    wv4 = wv.reshape(K, NS, NB, 128)

    deg = _p1_deg(si4, wv4).reshape(K, N2)
    dis2 = _p2_dis(deg)
    nrmT, gfT, siT = _p3_norm(gi4, si4, wv4, dis2.reshape(K * N2))
    nrmF = nrmT.reshape(NS, K * NB, 128)
    gfF = gfT.reshape(NS, K * NB, 128)
    siF = siT.reshape(NS, K * NB, 128)

    h = x
    for l in range(NL):
        wl = jnp.stack([W_in[l], W_out[l]], axis=1).reshape(K, D, H)
        hw = _k4_matmul(h, wl).reshape(NC * K * N, FH)
        acc2 = _k5_agg(hw, gfF, siF, nrmF)
        h = _k6_ln(acc2, b_in[l], b_out[l], gamma[l][None], beta[l][None])
    return h


# speedup vs baseline: 11.2866x; 1.0258x over previous
import functools

import jax
import jax.numpy as jnp
from jax import lax
from jax.experimental import pallas as pl
from jax.experimental.pallas import tpu as pltpu
from jax.experimental.pallas import tpu_sc as plsc

N = 10000
E = 160000
R = 8
D = 256
H = 256
NL = 3
ALPHA = 0.75

K = 2 * R
NS = 16
NC = 2
NB = 84
SB = 8
CHUNK = NB * 128
ETP = NS * CHUNK
N2 = 10240
FH = 128

_mesh = plsc.VectorSubcoreMesh(core_axis_name="c", subcore_axis_name="s")


def _zero_vmem(ref, nwords):
    zeros = jnp.zeros((16,), ref.dtype)

    def body(i, _):
        ref[pl.ds(i * 16, 16)] = zeros
        return 0

    lax.fori_loop(0, nwords // 16, body, 0)


def _zero_vmem2(ref, m, w):
    zeros = jnp.zeros((16,), ref.dtype)

    def body(i, _):
        for q in range(w // 16):
            ref[i, pl.ds(q * 16, 16)] = zeros
        return 0

    lax.fori_loop(0, m, body, 0)


def _bcast_lane(v, t):
    idx = jnp.full((16,), t, jnp.int32)
    return lax.gather(
        v, idx[:, None],
        dimension_numbers=lax.GatherDimensionNumbers(
            offset_dims=(), collapsed_slice_dims=(0,), start_index_map=(0,)),
        slice_sizes=(1,),
        mode=lax.GatherScatterMode.PROMISE_IN_BOUNDS)


@functools.partial(
    pl.kernel,
    out_type=jax.ShapeDtypeStruct((NC, (K // NC) * N2), jnp.float32),
    mesh=_mesh,
    scratch_types=[
        pltpu.VMEM_SHARED(((K // NC) * N2,), jnp.float32),
        pltpu.VMEM((5120,), jnp.float32),
        pltpu.VMEM((NB, 128), jnp.int32),
        pltpu.VMEM((NB, 128), jnp.float32),
    ],
)
def _p1_deg(si_hbm, wv_hbm, out_hbm, deg_sh, zbuf, si_v, wv_v):
    c = lax.axis_index("c")
    s = lax.axis_index("s")
    _zero_vmem(zbuf, 5120)
    pltpu.sync_copy(zbuf, deg_sh.at[pl.ds(s * 5120, 5120)])
    plsc.subcore_barrier()

    def per_k(kk, _):
        k = c * (K // NC) + kk
        pltpu.sync_copy(si_hbm.at[k, s], si_v)
        pltpu.sync_copy(wv_hbm.at[k, s], wv_v)
        off = kk * N2

        def add_off(b, _):
            for q in range(8):
                si_v[b, pl.ds(q * 16, 16)] = si_v[b, pl.ds(q * 16, 16)] + off
            return 0

        lax.fori_loop(0, NB, add_off, 0)

        def scat(b, _):
            pltpu.sync_copy(wv_v.at[b], deg_sh.at[si_v.at[b]], add=True)
            return 0

        lax.fori_loop(0, NB, scat, 0)
        return 0

    lax.fori_loop(0, K // NC, per_k, 0)
    plsc.subcore_barrier()

    @pl.when(s == 0)
    def _():
        pltpu.sync_copy(deg_sh, out_hbm.at[c])


def _p2_body(deg_ref, out_ref):
    d = deg_ref[...]
    dis = jnp.where(d > 0.0, lax.rsqrt(d), 0.0)
    rows = lax.broadcasted_iota(jnp.int32, (K, N2), 0)
    a = jnp.where(rows % 2 == 0, ALPHA ** 0.5, (1.0 - ALPHA) ** 0.5)
    out_ref[...] = dis * a


_p2_dis = pl.pallas_call(
    _p2_body,
    out_shape=jax.ShapeDtypeStruct((K, N2), jnp.float32),
)


@functools.partial(
    pl.kernel,
    out_type=(
        jax.ShapeDtypeStruct((NS, K, NB, 128), jnp.float32),
        jax.ShapeDtypeStruct((NS, K, NB, 128), jnp.int32),
        jax.ShapeDtypeStruct((NS, K, NB, 128), jnp.int32),
    ),
    mesh=_mesh,
    scratch_types=[
        pltpu.VMEM((NB, 128), jnp.int32),
        pltpu.VMEM((NB, 128), jnp.int32),
        pltpu.VMEM((NB, 128), jnp.float32),
        pltpu.VMEM((NB, 128), jnp.float32),
        pltpu.VMEM((NB, 128), jnp.int32),
        pltpu.VMEM((2, 128), jnp.float32),
        pltpu.VMEM((2, 128), jnp.float32),
        pltpu.VMEM_SHARED((K * N2,), jnp.float32),
        pltpu.SemaphoreType.DMA,
    ],
)
def _p3_norm(gi_hbm, si_hbm, wv_hbm, dis_hbm, nrm_hbm, gf_hbm, sit_hbm,
             gi_v, si_v, wv_v, nrm_v, gf_v, dgb, dsb, dis_sh, dsem):
    c = lax.axis_index("c")
    s = lax.axis_index("s")

    @pl.when(s == 0)
    def _():
        pltpu.sync_copy(dis_hbm, dis_sh)

    plsc.subcore_barrier()

    def per_k(kk, _):
        k = c * (K // NC) + kk
        pltpu.sync_copy(gi_hbm.at[k, s], gi_v)
        pltpu.sync_copy(si_hbm.at[k, s], si_v)
        pltpu.sync_copy(wv_hbm.at[k, s], wv_v)
        pltpu.sync_copy(si_v, sit_hbm.at[s, k])
        goff = k * N
        doff = k * N2

        def add_off(b, _):
            for q in range(8):
                sl = pl.ds(q * 16, 16)
                g = gi_v[b, sl]
                gf_v[b, sl] = g + goff
                gi_v[b, sl] = g + doff
                si_v[b, sl] = si_v[b, sl] + doff
            return 0

        lax.fori_loop(0, NB, add_off, 0)

        pltpu.async_copy(dis_sh.at[gi_v.at[0]], dgb.at[0], dsem)
        pltpu.async_copy(dis_sh.at[si_v.at[0]], dsb.at[0], dsem)

        def per_batch(b, _):
            sb = lax.rem(b, 2)
            nsb = lax.rem(b + 1, 2)
            pltpu.make_async_copy(dis_sh.at[gi_v.at[b]], dgb.at[sb],
                                  dsem).wait()
            pltpu.make_async_copy(dis_sh.at[si_v.at[b]], dsb.at[sb],
                                  dsem).wait()

            @pl.when(b + 1 < NB)
            def _():
                pltpu.async_copy(dis_sh.at[gi_v.at[b + 1]], dgb.at[nsb], dsem)
                pltpu.async_copy(dis_sh.at[si_v.at[b + 1]], dsb.at[nsb], dsem)

            for q in range(8):
                sl = pl.ds(q * 16, 16)
                nrm_v[b, sl] = dgb[sb, sl] * wv_v[b, sl] * dsb[sb, sl]
            return 0

        lax.fori_loop(0, NB, per_batch, 0)
        pltpu.sync_copy(nrm_v, nrm_hbm.at[s, k])
        pltpu.sync_copy(gf_v, gf_hbm.at[s, k])
        return 0

    lax.fori_loop(0, K // NC, per_k, 0)


def _k4_body(x_ref, w_ref, out_ref):
    out_ref[0, 0] = jnp.dot(x_ref[...], w_ref[0],
                            preferred_element_type=jnp.float32)


BN = 1000

_k4_matmul = pl.pallas_call(
    _k4_body,
    grid=(N // BN, K, NC),
    in_specs=[
        pl.BlockSpec((BN, D), lambda nb, k, c: (nb, 0)),
        pl.BlockSpec((1, D, FH), lambda nb, k, c: (k, 0, c)),
    ],
    out_specs=pl.BlockSpec((1, 1, BN, FH), lambda nb, k, c: (c, k, nb, 0)),
    out_shape=jax.ShapeDtypeStruct((NC, K, N, FH), jnp.float32),
)


NSTT = K * NB // SB


@functools.partial(
    pl.kernel,
    out_type=jax.ShapeDtypeStruct((NC, N, FH), jnp.float32),
    mesh=_mesh,
    scratch_types=[
        pltpu.VMEM_SHARED((N, FH), jnp.float32),
        pltpu.VMEM((2, 128, FH), jnp.float32),
        pltpu.VMEM((2, SB, 128), jnp.int32),
        pltpu.VMEM((2, SB, 128), jnp.int32),
        pltpu.VMEM((2, SB, 128), jnp.float32),
        pltpu.SemaphoreType.DMA,
        pltpu.SemaphoreType.DMA,
        pltpu.SemaphoreType.DMA,
    ],
)
def _k5_agg(hw_hbm, gf_hbm, si_hbm, nrm_hbm, out_hbm,
            acc_sh, rows, gf_v, si_v, nr_v, gsem, ssem, stsem):
    c = lax.axis_index("c")
    s = lax.axis_index("s")

    _zero_vmem2(rows.at[0], 128, FH)
    for i in range(5):
        j = s + 16 * i

        @pl.when(j < 78)
        def _():
            pltpu.sync_copy(rows.at[0], acc_sh.at[pl.ds(j * 128, 128)])

    @pl.when(s == 15)
    def _():
        pltpu.sync_copy(rows.at[0, pl.ds(0, 16)], acc_sh.at[pl.ds(9984, 16)])

    plsc.subcore_barrier()

    coff = c * (K * N)

    pltpu.async_copy(gf_hbm.at[s, pl.ds(0, SB)], gf_v.at[0], stsem)
    pltpu.async_copy(si_hbm.at[s, pl.ds(0, SB)], si_v.at[0], stsem)
    pltpu.async_copy(nrm_hbm.at[s, pl.ds(0, SB)], nr_v.at[0], stsem)

    def per_slice(st, _):
        isl = lax.rem(st, 2)
        insl = lax.rem(st + 1, 2)
        pltpu.make_async_copy(gf_hbm.at[s, pl.ds(0, SB)], gf_v.at[isl],
                              stsem).wait()
        pltpu.make_async_copy(si_hbm.at[s, pl.ds(0, SB)], si_v.at[isl],
                              stsem).wait()
        pltpu.make_async_copy(nrm_hbm.at[s, pl.ds(0, SB)], nr_v.at[isl],
                              stsem).wait()

        @pl.when(st + 1 < NSTT)
        def _():
            b1 = (st + 1) * SB
            pltpu.async_copy(gf_hbm.at[s, pl.ds(b1, SB)], gf_v.at[insl],
                             stsem)
            pltpu.async_copy(si_hbm.at[s, pl.ds(b1, SB)], si_v.at[insl],
                             stsem)
            pltpu.async_copy(nrm_hbm.at[s, pl.ds(b1, SB)], nr_v.at[insl],
                             stsem)

        def add_off(b, _):
            for q in range(8):
                sl = pl.ds(q * 16, 16)
                gf_v[isl, b, sl] = gf_v[isl, b, sl] + coff
            return 0

        lax.fori_loop(0, SB, add_off, 0)

        pltpu.async_copy(hw_hbm.at[gf_v.at[isl, 0]], rows.at[0], gsem)

        def per_batch(b, _):
            slot = lax.rem(b, 2)
            nslot = lax.rem(b + 1, 2)
            pltpu.make_async_copy(hw_hbm.at[gf_v.at[isl, b]], rows.at[slot],
                                  gsem).wait()

            @pl.when(b >= 1)
            def _():
                pltpu.make_async_copy(rows.at[nslot],
                                      acc_sh.at[si_v.at[isl, b]], ssem).wait()

            @pl.when(b + 1 < SB)
            def _():
                pltpu.async_copy(hw_hbm.at[gf_v.at[isl, b + 1]],
                                 rows.at[nslot], gsem)

            def scale_q(q, _):
                nrm16 = nr_v[isl, b, pl.ds(q * 16, 16)]

                def scale_t(t, _):
                    for dt in range(2):
                        e = q * 16 + t * 2 + dt
                        nv = _bcast_lane(nrm16, t * 2 + dt)
                        for f in range(8):
                            sl = pl.ds(f * 16, 16)
                            rows[slot, e, sl] = rows[slot, e, sl] * nv
                    return 0

                lax.fori_loop(0, 8, scale_t, 0)
                return 0

            lax.fori_loop(0, 8, scale_q, 0)
            pltpu.async_copy(rows.at[slot], acc_sh.at[si_v.at[isl, b]], ssem,
                             add=True)
            return 0

        lax.fori_loop(0, SB, per_batch, 0)
        pltpu.make_async_copy(rows.at[0], acc_sh.at[si_v.at[isl, 0]],
                              ssem).wait()
        return 0

    lax.fori_loop(0, NSTT, per_slice, 0)
    plsc.subcore_barrier()
    base = s * 624
    pltpu.sync_copy(acc_sh.at[pl.ds(base, 624)], out_hbm.at[c, pl.ds(base, 624)])

    @pl.when(s == 15)
    def _():
        pltpu.sync_copy(acc_sh.at[pl.ds(9984, 16)],
                        out_hbm.at[c, pl.ds(9984, 16)])


def _k6_body(acc_ref, bi_ref, bo_ref, g_ref, bt_ref, out_ref):
    z = jnp.concatenate([acc_ref[0], acc_ref[1]], axis=1)
    bsum = (jnp.sum(bi_ref[...], axis=0, keepdims=True) * ALPHA
            + jnp.sum(bo_ref[...], axis=0, keepdims=True) * (1.0 - ALPHA))
    z = z + bsum
    mu = jnp.mean(z, axis=-1, keepdims=True)
    zc = z - mu
    var = jnp.mean(zc * zc, axis=-1, keepdims=True)
    y = zc * lax.rsqrt(var + 1e-5) * g_ref[...] + bt_ref[...]
    out_ref[...] = jnp.maximum(y, 0.0)


_k6_ln = pl.pallas_call(
    _k6_body,
    grid=(N // BN,),
    in_specs=[
        pl.BlockSpec((NC, BN, FH), lambda nb: (0, nb, 0)),
        pl.BlockSpec((R, H), lambda nb: (0, 0)),
        pl.BlockSpec((R, H), lambda nb: (0, 0)),
        pl.BlockSpec((1, H), lambda nb: (0, 0)),
        pl.BlockSpec((1, H), lambda nb: (0, 0)),
    ],
    out_specs=pl.BlockSpec((BN, H), lambda nb: (nb, 0)),
    out_shape=jax.ShapeDtypeStruct((N, H), jnp.float32),
)


def kernel(x, edge_index, edge_weight, W_in, b_in, W_out, b_out, gamma, beta):
    pad = ETP - E - N
    loops_i = jnp.broadcast_to(jnp.arange(N, dtype=jnp.int32), (K, N))
    pad_i = jnp.broadcast_to(jnp.arange(pad, dtype=jnp.int32) % N, (K, pad))
    gi = jnp.concatenate([edge_index.reshape(K, E), loops_i, pad_i], axis=1)
    si = jnp.concatenate(
        [edge_index[:, ::-1, :].reshape(K, E), loops_i, pad_i], axis=1)
    wv = jnp.concatenate(
        [jnp.repeat(edge_weight, 2, axis=0), jnp.ones((K, N), jnp.float32),
         jnp.zeros((K, pad), jnp.float32)], axis=1)
    gi4 = gi.reshape(K, NS, NB, 128)
    si4 = si.reshape(K, NS, NB, 128)
    wv4 = wv.reshape(K, NS, NB, 128)

    deg = _p1_deg(si4, wv4).reshape(K, N2)
    dis2 = _p2_dis(deg)
    nrmT, gfT, siT = _p3_norm(gi4, si4, wv4, dis2.reshape(K * N2))
    nrmF = nrmT.reshape(NS, K * NB, 128)
    gfF = gfT.reshape(NS, K * NB, 128)
    siF = siT.reshape(NS, K * NB, 128)

    h = x
    for l in range(NL):
        wl = jnp.stack([W_in[l], W_out[l]], axis=1).reshape(K, D, H)
        hw = _k4_matmul(h, wl).reshape(NC * K * N, FH)
        acc2 = _k5_agg(hw, gfF, siF, nrmF)
        h = _k6_ln(acc2, b_in[l], b_out[l], gamma[l][None], beta[l][None])
    return h


# speedup vs baseline: 11.7823x; 1.0439x over previous
import functools

import jax
import jax.numpy as jnp
from jax import lax
from jax.experimental import pallas as pl
from jax.experimental.pallas import tpu as pltpu
from jax.experimental.pallas import tpu_sc as plsc

N = 10000
E = 160000
R = 8
D = 256
H = 256
NL = 3
ALPHA = 0.75

K = 2 * R
NS = 16
NC = 2
NB = 84
NB2 = 79
SB = 8
CHUNK = NB * 128
ETP = NS * CHUNK
CHUNK2 = NB2 * 128
ETP2 = NS * CHUNK2
N2 = 10240
FH = 128

_mesh = plsc.VectorSubcoreMesh(core_axis_name="c", subcore_axis_name="s")


def _zero_vmem(ref, nwords):
    zeros = jnp.zeros((16,), ref.dtype)

    def body(i, _):
        ref[pl.ds(i * 16, 16)] = zeros
        return 0

    lax.fori_loop(0, nwords // 16, body, 0)


def _zero_vmem2(ref, m, w):
    zeros = jnp.zeros((16,), ref.dtype)

    def body(i, _):
        for q in range(w // 16):
            ref[i, pl.ds(q * 16, 16)] = zeros
        return 0

    lax.fori_loop(0, m, body, 0)


def _bcast_lane(v, t):
    idx = jnp.full((16,), t, jnp.int32)
    return lax.gather(
        v, idx[:, None],
        dimension_numbers=lax.GatherDimensionNumbers(
            offset_dims=(), collapsed_slice_dims=(0,), start_index_map=(0,)),
        slice_sizes=(1,),
        mode=lax.GatherScatterMode.PROMISE_IN_BOUNDS)


@functools.partial(
    pl.kernel,
    out_type=jax.ShapeDtypeStruct((NC, (K // NC) * N2), jnp.float32),
    mesh=_mesh,
    scratch_types=[
        pltpu.VMEM_SHARED(((K // NC) * N2,), jnp.float32),
        pltpu.VMEM((5120,), jnp.float32),
        pltpu.VMEM((NB, 128), jnp.int32),
        pltpu.VMEM((NB, 128), jnp.float32),
    ],
)
def _p1_deg(si_hbm, wv_hbm, out_hbm, deg_sh, zbuf, si_v, wv_v):
    c = lax.axis_index("c")
    s = lax.axis_index("s")
    _zero_vmem(zbuf, 5120)
    pltpu.sync_copy(zbuf, deg_sh.at[pl.ds(s * 5120, 5120)])
    plsc.subcore_barrier()

    def per_k(kk, _):
        k = c * (K // NC) + kk
        pltpu.sync_copy(si_hbm.at[k, s], si_v)
        pltpu.sync_copy(wv_hbm.at[k, s], wv_v)
        off = kk * N2

        def add_off(b, _):
            for q in range(8):
                si_v[b, pl.ds(q * 16, 16)] = si_v[b, pl.ds(q * 16, 16)] + off
            return 0

        lax.fori_loop(0, NB, add_off, 0)

        def scat(b, _):
            pltpu.sync_copy(wv_v.at[b], deg_sh.at[si_v.at[b]], add=True)
            return 0

        lax.fori_loop(0, NB, scat, 0)
        return 0

    lax.fori_loop(0, K // NC, per_k, 0)
    plsc.subcore_barrier()

    @pl.when(s == 0)
    def _():
        pltpu.sync_copy(deg_sh, out_hbm.at[c])


def _p2_body(deg_ref, out_ref, dia_ref):
    d = deg_ref[...]
    dis = jnp.where(d > 0.0, lax.rsqrt(d), 0.0)
    rows = lax.broadcasted_iota(jnp.int32, (K, N2), 0)
    a = jnp.where(rows % 2 == 0, ALPHA ** 0.5, (1.0 - ALPHA) ** 0.5)
    dis2 = dis * a
    out_ref[...] = dis2
    dia_ref[...] = dis2 * dis2


_p2_dis = pl.pallas_call(
    _p2_body,
    out_shape=(jax.ShapeDtypeStruct((K, N2), jnp.float32),
               jax.ShapeDtypeStruct((K, N2), jnp.float32)),
)


@functools.partial(
    pl.kernel,
    out_type=(
        jax.ShapeDtypeStruct((NS, K, NB2, 128), jnp.float32),
        jax.ShapeDtypeStruct((NS, K, NB2, 128), jnp.int32),
        jax.ShapeDtypeStruct((NS, K, NB2, 128), jnp.int32),
    ),
    mesh=_mesh,
    scratch_types=[
        pltpu.VMEM((NB2, 128), jnp.int32),
        pltpu.VMEM((NB2, 128), jnp.int32),
        pltpu.VMEM((NB2, 128), jnp.float32),
        pltpu.VMEM((NB2, 128), jnp.float32),
        pltpu.VMEM((NB2, 128), jnp.int32),
        pltpu.VMEM((2, 128), jnp.float32),
        pltpu.VMEM((2, 128), jnp.float32),
        pltpu.VMEM_SHARED((K * N2,), jnp.float32),
        pltpu.SemaphoreType.DMA,
    ],
)
def _p3_norm(gi_hbm, si_hbm, wv_hbm, dis_hbm, nrm_hbm, gf_hbm, sit_hbm,
             gi_v, si_v, wv_v, nrm_v, gf_v, dgb, dsb, dis_sh, dsem):
    c = lax.axis_index("c")
    s = lax.axis_index("s")

    @pl.when(s == 0)
    def _():
        pltpu.sync_copy(dis_hbm, dis_sh)

    plsc.subcore_barrier()

    def per_k(kk, _):
        k = c * (K // NC) + kk
        pltpu.sync_copy(gi_hbm.at[k, s], gi_v)
        pltpu.sync_copy(si_hbm.at[k, s], si_v)
        pltpu.sync_copy(wv_hbm.at[k, s], wv_v)
        pltpu.sync_copy(si_v, sit_hbm.at[s, k])
        goff = k * N
        doff = k * N2

        def add_off(b, _):
            for q in range(8):
                sl = pl.ds(q * 16, 16)
                g = gi_v[b, sl]
                gf_v[b, sl] = g + goff
                gi_v[b, sl] = g + doff
                si_v[b, sl] = si_v[b, sl] + doff
            return 0

        lax.fori_loop(0, NB2, add_off, 0)

        pltpu.async_copy(dis_sh.at[gi_v.at[0]], dgb.at[0], dsem)
        pltpu.async_copy(dis_sh.at[si_v.at[0]], dsb.at[0], dsem)

        def per_batch(b, _):
            sb = lax.rem(b, 2)
            nsb = lax.rem(b + 1, 2)
            pltpu.make_async_copy(dis_sh.at[gi_v.at[b]], dgb.at[sb],
                                  dsem).wait()
            pltpu.make_async_copy(dis_sh.at[si_v.at[b]], dsb.at[sb],
                                  dsem).wait()

            @pl.when(b + 1 < NB2)
            def _():
                pltpu.async_copy(dis_sh.at[gi_v.at[b + 1]], dgb.at[nsb], dsem)
                pltpu.async_copy(dis_sh.at[si_v.at[b + 1]], dsb.at[nsb], dsem)

            for q in range(8):
                sl = pl.ds(q * 16, 16)
                nrm_v[b, sl] = dgb[sb, sl] * wv_v[b, sl] * dsb[sb, sl]
            return 0

        lax.fori_loop(0, NB2, per_batch, 0)
        pltpu.sync_copy(nrm_v, nrm_hbm.at[s, k])
        pltpu.sync_copy(gf_v, gf_hbm.at[s, k])
        return 0

    lax.fori_loop(0, K // NC, per_k, 0)


def _k4_body(x_ref, w_ref, dia_ref, out_ref, sl_ref):
    y = jnp.dot(x_ref[...], w_ref[0], preferred_element_type=jnp.float32)
    out_ref[0, 0] = y
    d = dia_ref[0, 0, 0][:, None]

    @pl.when(pl.program_id(2) == 0)
    def _():
        sl_ref[0] = d * y

    @pl.when(pl.program_id(2) > 0)
    def _():
        sl_ref[0] = sl_ref[0] + d * y


BN = 1000

_k4_matmul = pl.pallas_call(
    _k4_body,
    grid=(N // BN, NC, K),
    in_specs=[
        pl.BlockSpec((BN, D), lambda nb, c, k: (nb, 0)),
        pl.BlockSpec((1, D, FH), lambda nb, c, k: (k, 0, c)),
        pl.BlockSpec((1, 1, 1, BN), lambda nb, c, k: (k, nb, 0, 0)),
    ],
    out_specs=(
        pl.BlockSpec((1, 1, BN, FH), lambda nb, c, k: (c, k, nb, 0)),
        pl.BlockSpec((1, BN, FH), lambda nb, c, k: (c, nb, 0)),
    ),
    out_shape=(jax.ShapeDtypeStruct((NC, K, N, FH), jnp.float32),
               jax.ShapeDtypeStruct((NC, N, FH), jnp.float32)),
)


NSTT = K * NB2 // SB


@functools.partial(
    pl.kernel,
    out_type=jax.ShapeDtypeStruct((NC, N, FH), jnp.float32),
    mesh=_mesh,
    scratch_types=[
        pltpu.VMEM_SHARED((N, FH), jnp.float32),
        pltpu.VMEM((2, 128, FH), jnp.float32),
        pltpu.VMEM((2, SB, 128), jnp.int32),
        pltpu.VMEM((2, SB, 128), jnp.int32),
        pltpu.VMEM((2, SB, 128), jnp.float32),
        pltpu.SemaphoreType.DMA,
        pltpu.SemaphoreType.DMA,
        pltpu.SemaphoreType.DMA,
    ],
)
def _k5_agg(hw_hbm, gf_hbm, si_hbm, nrm_hbm, out_hbm,
            acc_sh, rows, gf_v, si_v, nr_v, gsem, ssem, stsem):
    c = lax.axis_index("c")
    s = lax.axis_index("s")

    _zero_vmem2(rows.at[0], 128, FH)
    for i in range(5):
        j = s + 16 * i

        @pl.when(j < 78)
        def _():
            pltpu.sync_copy(rows.at[0], acc_sh.at[pl.ds(j * 128, 128)])

    @pl.when(s == 15)
    def _():
        pltpu.sync_copy(rows.at[0, pl.ds(0, 16)], acc_sh.at[pl.ds(9984, 16)])

    plsc.subcore_barrier()

    coff = c * (K * N)

    pltpu.async_copy(gf_hbm.at[s, pl.ds(0, SB)], gf_v.at[0], stsem)
    pltpu.async_copy(si_hbm.at[s, pl.ds(0, SB)], si_v.at[0], stsem)
    pltpu.async_copy(nrm_hbm.at[s, pl.ds(0, SB)], nr_v.at[0], stsem)

    def per_slice(st, _):
        isl = lax.rem(st, 2)
        insl = lax.rem(st + 1, 2)
        pltpu.make_async_copy(gf_hbm.at[s, pl.ds(0, SB)], gf_v.at[isl],
                              stsem).wait()
        pltpu.make_async_copy(si_hbm.at[s, pl.ds(0, SB)], si_v.at[isl],
                              stsem).wait()
        pltpu.make_async_copy(nrm_hbm.at[s, pl.ds(0, SB)], nr_v.at[isl],
                              stsem).wait()

        @pl.when(st + 1 < NSTT)
        def _():
            b1 = (st + 1) * SB
            pltpu.async_copy(gf_hbm.at[s, pl.ds(b1, SB)], gf_v.at[insl],
                             stsem)
            pltpu.async_copy(si_hbm.at[s, pl.ds(b1, SB)], si_v.at[insl],
                             stsem)
            pltpu.async_copy(nrm_hbm.at[s, pl.ds(b1, SB)], nr_v.at[insl],
                             stsem)

        def add_off(b, _):
            for q in range(8):
                sl = pl.ds(q * 16, 16)
                gf_v[isl, b, sl] = gf_v[isl, b, sl] + coff
            return 0

        lax.fori_loop(0, SB, add_off, 0)

        pltpu.async_copy(hw_hbm.at[gf_v.at[isl, 0]], rows.at[0], gsem)

        def per_batch(b, _):
            slot = lax.rem(b, 2)
            nslot = lax.rem(b + 1, 2)
            pltpu.make_async_copy(hw_hbm.at[gf_v.at[isl, b]], rows.at[slot],
                                  gsem).wait()

            @pl.when(b >= 1)
            def _():
                pltpu.make_async_copy(rows.at[nslot],
                                      acc_sh.at[si_v.at[isl, b]], ssem).wait()

            @pl.when(b + 1 < SB)
            def _():
                pltpu.async_copy(hw_hbm.at[gf_v.at[isl, b + 1]],
                                 rows.at[nslot], gsem)

            def scale_q(q, _):
                nrm16 = nr_v[isl, b, pl.ds(q * 16, 16)]

                def scale_t(t, _):
                    for dt in range(2):
                        e = q * 16 + t * 2 + dt
                        nv = _bcast_lane(nrm16, t * 2 + dt)
                        for f in range(8):
                            sl = pl.ds(f * 16, 16)
                            rows[slot, e, sl] = rows[slot, e, sl] * nv
                    return 0

                lax.fori_loop(0, 8, scale_t, 0)
                return 0

            lax.fori_loop(0, 8, scale_q, 0)
            pltpu.async_copy(rows.at[slot], acc_sh.at[si_v.at[isl, b]], ssem,
                             add=True)
            return 0

        lax.fori_loop(0, SB, per_batch, 0)
        pltpu.make_async_copy(rows.at[0], acc_sh.at[si_v.at[isl, 0]],
                              ssem).wait()
        return 0

    lax.fori_loop(0, NSTT, per_slice, 0)
    plsc.subcore_barrier()
    base = s * 624
    pltpu.sync_copy(acc_sh.at[pl.ds(base, 624)], out_hbm.at[c, pl.ds(base, 624)])

    @pl.when(s == 15)
    def _():
        pltpu.sync_copy(acc_sh.at[pl.ds(9984, 16)],
                        out_hbm.at[c, pl.ds(9984, 16)])


def _k6_body(acc_ref, sl_ref, bi_ref, bo_ref, g_ref, bt_ref, out_ref):
    z = jnp.concatenate([acc_ref[0] + sl_ref[0], acc_ref[1] + sl_ref[1]],
                        axis=1)
    bsum = (jnp.sum(bi_ref[...], axis=0, keepdims=True) * ALPHA
            + jnp.sum(bo_ref[...], axis=0, keepdims=True) * (1.0 - ALPHA))
    z = z + bsum
    mu = jnp.mean(z, axis=-1, keepdims=True)
    zc = z - mu
    var = jnp.mean(zc * zc, axis=-1, keepdims=True)
    y = zc * lax.rsqrt(var + 1e-5) * g_ref[...] + bt_ref[...]
    out_ref[...] = jnp.maximum(y, 0.0)


_k6_ln = pl.pallas_call(
    _k6_body,
    grid=(N // BN,),
    in_specs=[
        pl.BlockSpec((NC, BN, FH), lambda nb: (0, nb, 0)),
        pl.BlockSpec((NC, BN, FH), lambda nb: (0, nb, 0)),
        pl.BlockSpec((R, H), lambda nb: (0, 0)),
        pl.BlockSpec((R, H), lambda nb: (0, 0)),
        pl.BlockSpec((1, H), lambda nb: (0, 0)),
        pl.BlockSpec((1, H), lambda nb: (0, 0)),
    ],
    out_specs=pl.BlockSpec((BN, H), lambda nb: (nb, 0)),
    out_shape=jax.ShapeDtypeStruct((N, H), jnp.float32),
)


def kernel(x, edge_index, edge_weight, W_in, b_in, W_out, b_out, gamma, beta):
    pad = ETP - E - N
    loops_i = jnp.broadcast_to(jnp.arange(N, dtype=jnp.int32), (K, N))
    pad_i = jnp.broadcast_to(jnp.arange(pad, dtype=jnp.int32) % N, (K, pad))
    siA = jnp.concatenate(
        [edge_index[:, ::-1, :].reshape(K, E), loops_i, pad_i], axis=1)
    wvA = jnp.concatenate(
        [jnp.repeat(edge_weight, 2, axis=0), jnp.ones((K, N), jnp.float32),
         jnp.zeros((K, pad), jnp.float32)], axis=1)
    si4 = siA.reshape(K, NS, NB, 128)
    wv4 = wvA.reshape(K, NS, NB, 128)

    pad2 = ETP2 - E
    pad2_i = jnp.broadcast_to(jnp.arange(pad2, dtype=jnp.int32) % N, (K, pad2))
    giB = jnp.concatenate([edge_index.reshape(K, E), pad2_i], axis=1)
    siB = jnp.concatenate(
        [edge_index[:, ::-1, :].reshape(K, E), pad2_i], axis=1)
    wvB = jnp.concatenate(
        [jnp.repeat(edge_weight, 2, axis=0), jnp.zeros((K, pad2), jnp.float32)],
        axis=1)
    gi4b = giB.reshape(K, NS, NB2, 128)
    si4b = siB.reshape(K, NS, NB2, 128)
    wv4b = wvB.reshape(K, NS, NB2, 128)

    deg = _p1_deg(si4, wv4).reshape(K, N2)
    dis2, dia = _p2_dis(deg)
    nrmT, gfT, siT = _p3_norm(gi4b, si4b, wv4b, dis2.reshape(K * N2))
    nrmF = nrmT.reshape(NS, K * NB2, 128)
    gfF = gfT.reshape(NS, K * NB2, 128)
    siF = siT.reshape(NS, K * NB2, 128)
    diaN = dia[:, :N].reshape(K, N // BN, 1, BN)

    h = x
    for l in range(NL):
        wl = jnp.stack([W_in[l], W_out[l]], axis=1).reshape(K, D, H)
        hw, sl = _k4_matmul(h, wl, diaN)
        hw = hw.reshape(NC * K * N, FH)
        acc2 = _k5_agg(hw, gfF, siF, nrmF)
        h = _k6_ln(acc2, sl, b_in[l], b_out[l], gamma[l][None], beta[l][None])
    return h


# speedup vs baseline: 12.1400x; 1.0304x over previous
import functools

import jax
import jax.numpy as jnp
from jax import lax
from jax.experimental import pallas as pl
from jax.experimental.pallas import tpu as pltpu
from jax.experimental.pallas import tpu_sc as plsc

N = 10000
E = 160000
R = 8
D = 256
H = 256
NL = 3
ALPHA = 0.75

K = 2 * R
NS = 16
NC = 2
NB = 84
NB2 = 79
SB = 16
CHUNK = NB * 128
ETP = NS * CHUNK
CHUNK2 = NB2 * 128
ETP2 = NS * CHUNK2
N2 = 10240
FH = 128

_mesh = plsc.VectorSubcoreMesh(core_axis_name="c", subcore_axis_name="s")


def _zero_vmem(ref, nwords):
    zeros = jnp.zeros((16,), ref.dtype)

    def body(i, _):
        ref[pl.ds(i * 16, 16)] = zeros
        return 0

    lax.fori_loop(0, nwords // 16, body, 0)


def _zero_vmem2(ref, m, w):
    zeros = jnp.zeros((16,), ref.dtype)

    def body(i, _):
        for q in range(w // 16):
            ref[i, pl.ds(q * 16, 16)] = zeros
        return 0

    lax.fori_loop(0, m, body, 0)


def _bcast_lane(v, t):
    idx = jnp.full((16,), t, jnp.int32)
    return lax.gather(
        v, idx[:, None],
        dimension_numbers=lax.GatherDimensionNumbers(
            offset_dims=(), collapsed_slice_dims=(0,), start_index_map=(0,)),
        slice_sizes=(1,),
        mode=lax.GatherScatterMode.PROMISE_IN_BOUNDS)


@functools.partial(
    pl.kernel,
    out_type=jax.ShapeDtypeStruct((NC, (K // NC) * N2), jnp.float32),
    mesh=_mesh,
    scratch_types=[
        pltpu.VMEM_SHARED(((K // NC) * N2,), jnp.float32),
        pltpu.VMEM((5120,), jnp.float32),
        pltpu.VMEM((NB, 128), jnp.int32),
        pltpu.VMEM((NB, 128), jnp.float32),
    ],
)
def _p1_deg(si_hbm, wv_hbm, out_hbm, deg_sh, zbuf, si_v, wv_v):
    c = lax.axis_index("c")
    s = lax.axis_index("s")
    _zero_vmem(zbuf, 5120)
    pltpu.sync_copy(zbuf, deg_sh.at[pl.ds(s * 5120, 5120)])
    plsc.subcore_barrier()

    def per_k(kk, _):
        k = c * (K // NC) + kk
        pltpu.sync_copy(si_hbm.at[k, s], si_v)
        pltpu.sync_copy(wv_hbm.at[k, s], wv_v)
        off = kk * N2

        def add_off(b, _):
            for q in range(8):
                si_v[b, pl.ds(q * 16, 16)] = si_v[b, pl.ds(q * 16, 16)] + off
            return 0

        lax.fori_loop(0, NB, add_off, 0)

        def scat(b, _):
            pltpu.sync_copy(wv_v.at[b], deg_sh.at[si_v.at[b]], add=True)
            return 0

        lax.fori_loop(0, NB, scat, 0)
        return 0

    lax.fori_loop(0, K // NC, per_k, 0)
    plsc.subcore_barrier()

    @pl.when(s == 0)
    def _():
        pltpu.sync_copy(deg_sh, out_hbm.at[c])


def _p2_body(deg_ref, out_ref, dia_ref):
    d = deg_ref[...]
    dis = jnp.where(d > 0.0, lax.rsqrt(d), 0.0)
    rows = lax.broadcasted_iota(jnp.int32, (K, N2), 0)
    a = jnp.where(rows % 2 == 0, ALPHA ** 0.5, (1.0 - ALPHA) ** 0.5)
    dis2 = dis * a
    out_ref[...] = dis2
    dia_ref[...] = dis2 * dis2


_p2_dis = pl.pallas_call(
    _p2_body,
    out_shape=(jax.ShapeDtypeStruct((K, N2), jnp.float32),
               jax.ShapeDtypeStruct((K, N2), jnp.float32)),
)


@functools.partial(
    pl.kernel,
    out_type=(
        jax.ShapeDtypeStruct((NS, K, NB2, 128), jnp.float32),
        jax.ShapeDtypeStruct((NS, K, NB2, 128), jnp.int32),
        jax.ShapeDtypeStruct((NS, K, NB2, 128), jnp.int32),
    ),
    mesh=_mesh,
    scratch_types=[
        pltpu.VMEM((NB2, 128), jnp.int32),
        pltpu.VMEM((NB2, 128), jnp.int32),
        pltpu.VMEM((NB2, 128), jnp.float32),
        pltpu.VMEM((NB2, 128), jnp.float32),
        pltpu.VMEM((NB2, 128), jnp.int32),
        pltpu.VMEM((2, 128), jnp.float32),
        pltpu.VMEM((2, 128), jnp.float32),
        pltpu.VMEM_SHARED((K * N2,), jnp.float32),
        pltpu.SemaphoreType.DMA,
    ],
)
def _p3_norm(gi_hbm, si_hbm, wv_hbm, dis_hbm, nrm_hbm, gf_hbm, sit_hbm,
             gi_v, si_v, wv_v, nrm_v, gf_v, dgb, dsb, dis_sh, dsem):
    c = lax.axis_index("c")
    s = lax.axis_index("s")

    @pl.when(s == 0)
    def _():
        pltpu.sync_copy(dis_hbm, dis_sh)

    plsc.subcore_barrier()

    def per_k(kk, _):
        k = c * (K // NC) + kk
        pltpu.sync_copy(gi_hbm.at[k, s], gi_v)
        pltpu.sync_copy(si_hbm.at[k, s], si_v)
        pltpu.sync_copy(wv_hbm.at[k, s], wv_v)
        pltpu.sync_copy(si_v, sit_hbm.at[s, k])
        goff = k * N
        doff = k * N2

        def add_off(b, _):
            for q in range(8):
                sl = pl.ds(q * 16, 16)
                g = gi_v[b, sl]
                gf_v[b, sl] = g + goff
                gi_v[b, sl] = g + doff
                si_v[b, sl] = si_v[b, sl] + doff
            return 0

        lax.fori_loop(0, NB2, add_off, 0)

        pltpu.async_copy(dis_sh.at[gi_v.at[0]], dgb.at[0], dsem)
        pltpu.async_copy(dis_sh.at[si_v.at[0]], dsb.at[0], dsem)

        def per_batch(b, _):
            sb = lax.rem(b, 2)
            nsb = lax.rem(b + 1, 2)
            pltpu.make_async_copy(dis_sh.at[gi_v.at[b]], dgb.at[sb],
                                  dsem).wait()
            pltpu.make_async_copy(dis_sh.at[si_v.at[b]], dsb.at[sb],
                                  dsem).wait()

            @pl.when(b + 1 < NB2)
            def _():
                pltpu.async_copy(dis_sh.at[gi_v.at[b + 1]], dgb.at[nsb], dsem)
                pltpu.async_copy(dis_sh.at[si_v.at[b + 1]], dsb.at[nsb], dsem)

            for q in range(8):
                sl = pl.ds(q * 16, 16)
                nrm_v[b, sl] = dgb[sb, sl] * wv_v[b, sl] * dsb[sb, sl]
            return 0

        lax.fori_loop(0, NB2, per_batch, 0)
        pltpu.sync_copy(nrm_v, nrm_hbm.at[s, k])
        pltpu.sync_copy(gf_v, gf_hbm.at[s, k])
        return 0

    lax.fori_loop(0, K // NC, per_k, 0)


def _k4_body(x_ref, w_ref, dia_ref, out_ref, sl_ref):
    y = jnp.dot(x_ref[...], w_ref[0], preferred_element_type=jnp.float32)
    out_ref[0, 0] = y
    d = dia_ref[0, 0, 0][:, None]

    @pl.when(pl.program_id(2) == 0)
    def _():
        sl_ref[0] = d * y

    @pl.when(pl.program_id(2) > 0)
    def _():
        sl_ref[0] = sl_ref[0] + d * y


BN = 1000

_k4_matmul = pl.pallas_call(
    _k4_body,
    grid=(N // BN, NC, K),
    in_specs=[
        pl.BlockSpec((BN, D), lambda nb, c, k: (nb, 0)),
        pl.BlockSpec((1, D, FH), lambda nb, c, k: (k, 0, c)),
        pl.BlockSpec((1, 1, 1, BN), lambda nb, c, k: (k, nb, 0, 0)),
    ],
    out_specs=(
        pl.BlockSpec((1, 1, BN, FH), lambda nb, c, k: (c, k, nb, 0)),
        pl.BlockSpec((1, BN, FH), lambda nb, c, k: (c, nb, 0)),
    ),
    out_shape=(jax.ShapeDtypeStruct((NC, K, N, FH), jnp.float32),
               jax.ShapeDtypeStruct((NC, N, FH), jnp.float32)),
)


NSTT = K * NB2 // SB


@functools.partial(
    pl.kernel,
    out_type=jax.ShapeDtypeStruct((NC, N, FH), jnp.float32),
    mesh=_mesh,
    scratch_types=[
        pltpu.VMEM_SHARED((N, FH), jnp.float32),
        pltpu.VMEM((2, 128, FH), jnp.float32),
        pltpu.VMEM((2, SB, 128), jnp.int32),
        pltpu.VMEM((2, SB, 128), jnp.int32),
        pltpu.VMEM((2, SB, 128), jnp.float32),
        pltpu.SemaphoreType.DMA,
        pltpu.SemaphoreType.DMA,
        pltpu.SemaphoreType.DMA,
    ],
)
def _k5_agg(hw_hbm, gf_hbm, si_hbm, nrm_hbm, out_hbm,
            acc_sh, rows, gf_v, si_v, nr_v, gsem, ssem, stsem):
    c = lax.axis_index("c")
    s = lax.axis_index("s")

    _zero_vmem2(rows.at[0], 128, FH)
    for i in range(5):
        j = s + 16 * i

        @pl.when(j < 78)
        def _():
            pltpu.sync_copy(rows.at[0], acc_sh.at[pl.ds(j * 128, 128)])

    @pl.when(s == 15)
    def _():
        pltpu.sync_copy(rows.at[0, pl.ds(0, 16)], acc_sh.at[pl.ds(9984, 16)])

    plsc.subcore_barrier()

    coff = c * (K * N)

    pltpu.async_copy(gf_hbm.at[s, pl.ds(0, SB)], gf_v.at[0], stsem)
    pltpu.async_copy(si_hbm.at[s, pl.ds(0, SB)], si_v.at[0], stsem)
    pltpu.async_copy(nrm_hbm.at[s, pl.ds(0, SB)], nr_v.at[0], stsem)

    def per_slice(st, _):
        isl = lax.rem(st, 2)
        insl = lax.rem(st + 1, 2)
        pltpu.make_async_copy(gf_hbm.at[s, pl.ds(0, SB)], gf_v.at[isl],
                              stsem).wait()
        pltpu.make_async_copy(si_hbm.at[s, pl.ds(0, SB)], si_v.at[isl],
                              stsem).wait()
        pltpu.make_async_copy(nrm_hbm.at[s, pl.ds(0, SB)], nr_v.at[isl],
                              stsem).wait()

        @pl.when(st + 1 < NSTT)
        def _():
            b1 = (st + 1) * SB
            pltpu.async_copy(gf_hbm.at[s, pl.ds(b1, SB)], gf_v.at[insl],
                             stsem)
            pltpu.async_copy(si_hbm.at[s, pl.ds(b1, SB)], si_v.at[insl],
                             stsem)
            pltpu.async_copy(nrm_hbm.at[s, pl.ds(b1, SB)], nr_v.at[insl],
                             stsem)

        def add_off(b, _):
            for q in range(8):
                sl = pl.ds(q * 16, 16)
                gf_v[isl, b, sl] = gf_v[isl, b, sl] + coff
            return 0

        lax.fori_loop(0, SB, add_off, 0)

        pltpu.async_copy(hw_hbm.at[gf_v.at[isl, 0]], rows.at[0], gsem)

        def per_batch(b, _):
            slot = lax.rem(b, 2)
            nslot = lax.rem(b + 1, 2)
            pltpu.make_async_copy(hw_hbm.at[gf_v.at[isl, b]], rows.at[slot],
                                  gsem).wait()

            @pl.when(b >= 1)
            def _():
                pltpu.make_async_copy(rows.at[nslot],
                                      acc_sh.at[si_v.at[isl, b]], ssem).wait()

            @pl.when(b + 1 < SB)
            def _():
                pltpu.async_copy(hw_hbm.at[gf_v.at[isl, b + 1]],
                                 rows.at[nslot], gsem)

            def scale_q(q, _):
                nrm16 = nr_v[isl, b, pl.ds(q * 16, 16)]

                def scale_t(t, _):
                    for dt in range(2):
                        e = q * 16 + t * 2 + dt
                        nv = _bcast_lane(nrm16, t * 2 + dt)
                        for f in range(8):
                            sl = pl.ds(f * 16, 16)
                            rows[slot, e, sl] = rows[slot, e, sl] * nv
                    return 0

                lax.fori_loop(0, 8, scale_t, 0)
                return 0

            lax.fori_loop(0, 8, scale_q, 0)
            pltpu.async_copy(rows.at[slot], acc_sh.at[si_v.at[isl, b]], ssem,
                             add=True)
            return 0

        lax.fori_loop(0, SB, per_batch, 0)
        pltpu.make_async_copy(rows.at[0], acc_sh.at[si_v.at[isl, 0]],
                              ssem).wait()
        return 0

    lax.fori_loop(0, NSTT, per_slice, 0)
    plsc.subcore_barrier()
    base = s * 624
    pltpu.sync_copy(acc_sh.at[pl.ds(base, 624)], out_hbm.at[c, pl.ds(base, 624)])

    @pl.when(s == 15)
    def _():
        pltpu.sync_copy(acc_sh.at[pl.ds(9984, 16)],
                        out_hbm.at[c, pl.ds(9984, 16)])


def _k6_body(acc_ref, sl_ref, bi_ref, bo_ref, g_ref, bt_ref, out_ref):
    z = jnp.concatenate([acc_ref[0] + sl_ref[0], acc_ref[1] + sl_ref[1]],
                        axis=1)
    bsum = (jnp.sum(bi_ref[...], axis=0, keepdims=True) * ALPHA
            + jnp.sum(bo_ref[...], axis=0, keepdims=True) * (1.0 - ALPHA))
    z = z + bsum
    mu = jnp.mean(z, axis=-1, keepdims=True)
    zc = z - mu
    var = jnp.mean(zc * zc, axis=-1, keepdims=True)
    y = zc * lax.rsqrt(var + 1e-5) * g_ref[...] + bt_ref[...]
    out_ref[...] = jnp.maximum(y, 0.0)


_k6_ln = pl.pallas_call(
    _k6_body,
    grid=(N // BN,),
    in_specs=[
        pl.BlockSpec((NC, BN, FH), lambda nb: (0, nb, 0)),
        pl.BlockSpec((NC, BN, FH), lambda nb: (0, nb, 0)),
        pl.BlockSpec((R, H), lambda nb: (0, 0)),
        pl.BlockSpec((R, H), lambda nb: (0, 0)),
        pl.BlockSpec((1, H), lambda nb: (0, 0)),
        pl.BlockSpec((1, H), lambda nb: (0, 0)),
    ],
    out_specs=pl.BlockSpec((BN, H), lambda nb: (nb, 0)),
    out_shape=jax.ShapeDtypeStruct((N, H), jnp.float32),
)


def kernel(x, edge_index, edge_weight, W_in, b_in, W_out, b_out, gamma, beta):
    pad = ETP - E - N
    loops_i = jnp.broadcast_to(jnp.arange(N, dtype=jnp.int32), (K, N))
    pad_i = jnp.broadcast_to(jnp.arange(pad, dtype=jnp.int32) % N, (K, pad))
    siA = jnp.concatenate(
        [edge_index[:, ::-1, :].reshape(K, E), loops_i, pad_i], axis=1)
    wvA = jnp.concatenate(
        [jnp.repeat(edge_weight, 2, axis=0), jnp.ones((K, N), jnp.float32),
         jnp.zeros((K, pad), jnp.float32)], axis=1)
    si4 = siA.reshape(K, NS, NB, 128)
    wv4 = wvA.reshape(K, NS, NB, 128)

    pad2 = ETP2 - E
    pad2_i = jnp.broadcast_to(jnp.arange(pad2, dtype=jnp.int32) % N, (K, pad2))
    giB = jnp.concatenate([edge_index.reshape(K, E), pad2_i], axis=1)
    siB = jnp.concatenate(
        [edge_index[:, ::-1, :].reshape(K, E), pad2_i], axis=1)
    wvB = jnp.concatenate(
        [jnp.repeat(edge_weight, 2, axis=0), jnp.zeros((K, pad2), jnp.float32)],
        axis=1)
    gi4b = giB.reshape(K, NS, NB2, 128)
    si4b = siB.reshape(K, NS, NB2, 128)
    wv4b = wvB.reshape(K, NS, NB2, 128)

    deg = _p1_deg(si4, wv4).reshape(K, N2)
    dis2, dia = _p2_dis(deg)
    nrmT, gfT, siT = _p3_norm(gi4b, si4b, wv4b, dis2.reshape(K * N2))
    nrmF = nrmT.reshape(NS, K * NB2, 128)
    gfF = gfT.reshape(NS, K * NB2, 128)
    siF = siT.reshape(NS, K * NB2, 128)
    diaN = dia[:, :N].reshape(K, N // BN, 1, BN)

    h = x
    for l in range(NL):
        wl = jnp.stack([W_in[l], W_out[l]], axis=1).reshape(K, D, H)
        hw, sl = _k4_matmul(h, wl, diaN)
        hw = hw.reshape(NC * K * N, FH)
        acc2 = _k5_agg(hw, gfF, siF, nrmF)
        h = _k6_ln(acc2, sl, b_in[l], b_out[l], gamma[l][None], beta[l][None])
    return h
